# baseline (speedup 1.0000x reference)
import functools

import jax
import jax.numpy as jnp
from jax import lax
from jax.experimental import pallas as pl
from jax.experimental.pallas import tpu as pltpu
from jax.experimental.pallas import tpu_sc as plsc

B = 1024
L = 1024
VOCAB = 100
EMB = 100
HALF = EMB // 2
W32 = EMB // 2

GL = 128
NW = 32
ROWS = B * L
CHUNK = 1024
IDX_ROWS = CHUNK // 128


def _prep_body(inp_rows, inp_cols, table_ref, w_ref, b_ref, bt_ref, fidx_ref):
    i = pl.program_id(0)
    x = inp_rows[...].astype(jnp.float32).astype(jnp.bfloat16)
    d = jnp.dot(x, w_ref[...], preferred_element_type=jnp.float32)
    d = d.astype(jnp.bfloat16) + b_ref[...]
    t16 = table_ref[...].astype(jnp.bfloat16)
    s = t16[:, None, :] + d[None, :, :]
    u1 = lax.bitcast_convert_type(s[..., :HALF], jnp.uint16).astype(jnp.uint32)
    u2 = lax.bitcast_convert_type(s[..., HALF:], jnp.uint16).astype(jnp.uint32)
    bt_ref[...] = lax.bitcast_convert_type(u1 | (u2 << 16), jnp.int32)
    l_iota = i * GL + lax.broadcasted_iota(jnp.int32, (B, GL), 1)
    fidx_ref[...] = inp_cols[...] * L + l_iota


def _prep(inputs, table, W, b2):
    return pl.pallas_call(
        _prep_body,
        grid=(L // GL,),
        in_specs=[
            pl.BlockSpec((GL, L), lambda i: (i, 0)),
            pl.BlockSpec((B, GL), lambda i: (0, i)),
            pl.BlockSpec((VOCAB, EMB), lambda i: (0, 0)),
            pl.BlockSpec((L, EMB), lambda i: (0, 0)),
            pl.BlockSpec((1, EMB), lambda i: (0, 0)),
        ],
        out_specs=[
            pl.BlockSpec((VOCAB, GL, W32), lambda i: (0, i, 0)),
            pl.BlockSpec((B, GL), lambda i: (0, i)),
        ],
        out_shape=[
            jax.ShapeDtypeStruct((VOCAB, L, W32), jnp.int32),
            jax.ShapeDtypeStruct((B, L), jnp.int32),
        ],
    )(inputs, inputs, table, W, b2)


_sc_mesh = plsc.VectorSubcoreMesh(core_axis_name="c", subcore_axis_name="s")


@functools.partial(
    pl.kernel,
    mesh=_sc_mesh,
    out_type=jax.ShapeDtypeStruct((B, IDX_ROWS, 128, W32), jnp.int32),
    scratch_types=[
        pltpu.VMEM((IDX_ROWS, 128), jnp.int32),
        pltpu.VMEM((IDX_ROWS, 128, W32), jnp.int32),
        pltpu.SemaphoreType.DMA,
    ],
    compiler_params=pltpu.CompilerParams(use_tc_tiling_on_sc=False),
)
def _sc_gather(bt_hbm, fidx_hbm, out_hbm, idx_v, rows_v, sem):
    wid = lax.axis_index("s") * 2 + lax.axis_index("c")
    chunks_per_w = B // NW

    def body(c, carry):
        chunk = wid * chunks_per_w + c
        pltpu.sync_copy(fidx_hbm.at[pl.ds(chunk * IDX_ROWS, IDX_ROWS)], idx_v)
        handles = [
            pltpu.async_copy(bt_hbm.at[idx_v.at[j]], rows_v.at[j], sem)
            for j in range(IDX_ROWS)
        ]
        for h in handles:
            h.wait()
        pltpu.sync_copy(rows_v, out_hbm.at[chunk])
        return carry

    lax.fori_loop(0, chunks_per_w, body, 0)


BB = 8


def _epi_body(z_ref, o_ref):
    z = z_ref[...][:, 0]
    lo = lax.bitcast_convert_type(z << 16, jnp.float32).astype(jnp.bfloat16)
    hi = lax.bitcast_convert_type(
        z & jnp.int32(-65536), jnp.float32
    ).astype(jnp.bfloat16)
    o_ref[...] = jnp.concatenate([lo, hi], axis=-1)


def _epilogue(z):
    return pl.pallas_call(
        _epi_body,
        grid=(B // BB, IDX_ROWS),
        in_specs=[
            pl.BlockSpec((BB, 1, 128, W32), lambda i, j: (i, j, 0, 0))
        ],
        out_specs=pl.BlockSpec((BB, 128, EMB), lambda i, j: (i, j, 0)),
        out_shape=jax.ShapeDtypeStruct((B, L, EMB), jnp.bfloat16),
    )(z)


def kernel(inputs, table, W, b):
    btp, fidx = _prep(inputs, table, W, b.reshape(1, EMB))
    bt_i32 = btp.reshape(VOCAB * L, W32)
    fidx3 = fidx.reshape(ROWS // 128, 128)
    z = _sc_gather(bt_i32, fidx3)
    return _epilogue(z)


# speedup vs baseline: 1.4964x; 1.4964x over previous
import functools

import jax
import jax.numpy as jnp
from jax import lax
from jax.experimental import pallas as pl
from jax.experimental.pallas import tpu as pltpu
from jax.experimental.pallas import tpu_sc as plsc

B = 1024
L = 1024
VOCAB = 100
EMB = 100
HALF = EMB // 2
W32 = EMB // 2

GL = 128
NW = 32
ROWS = B * L
CHUNK = 1024
IDX_ROWS = CHUNK // 128


def _prep_body(inp_rows, inp_cols, table_ref, w_ref, b_ref, bt_ref, fidx_ref):
    i = pl.program_id(0)
    x = inp_rows[...].astype(jnp.float32).astype(jnp.bfloat16)
    d = jnp.dot(x, w_ref[...], preferred_element_type=jnp.float32)
    d = d.astype(jnp.bfloat16) + b_ref[...]
    t16 = table_ref[...].astype(jnp.bfloat16)
    s = t16[:, None, :] + d[None, :, :]
    u1 = lax.bitcast_convert_type(s[..., :HALF], jnp.uint16).astype(jnp.uint32)
    u2 = lax.bitcast_convert_type(s[..., HALF:], jnp.uint16).astype(jnp.uint32)
    bt_ref[...] = lax.bitcast_convert_type(u1 | (u2 << 16), jnp.int32)
    l_iota = i * GL + lax.broadcasted_iota(jnp.int32, (B, GL), 1)
    fidx_ref[...] = inp_cols[...] * L + l_iota


def _prep(inputs, table, W, b2):
    return pl.pallas_call(
        _prep_body,
        grid=(L // GL,),
        in_specs=[
            pl.BlockSpec((GL, L), lambda i: (i, 0)),
            pl.BlockSpec((B, GL), lambda i: (0, i)),
            pl.BlockSpec((VOCAB, EMB), lambda i: (0, 0)),
            pl.BlockSpec((L, EMB), lambda i: (0, 0)),
            pl.BlockSpec((1, EMB), lambda i: (0, 0)),
        ],
        out_specs=[
            pl.BlockSpec((VOCAB, GL, W32), lambda i: (0, i, 0)),
            pl.BlockSpec((B, GL), lambda i: (0, i)),
        ],
        out_shape=[
            jax.ShapeDtypeStruct((VOCAB, L, W32), jnp.int32),
            jax.ShapeDtypeStruct((B, L), jnp.int32),
        ],
    )(inputs, inputs, table, W, b2)


_sc_mesh = plsc.VectorSubcoreMesh(core_axis_name="c", subcore_axis_name="s")


@functools.partial(
    pl.kernel,
    mesh=_sc_mesh,
    out_type=jax.ShapeDtypeStruct((B, IDX_ROWS, 128, W32), jnp.int32),
    scratch_types=[
        pltpu.VMEM((IDX_ROWS, 128), jnp.int32),
        pltpu.VMEM((IDX_ROWS, 128, W32), jnp.int32),
        pltpu.SemaphoreType.DMA,
    ],
    compiler_params=pltpu.CompilerParams(use_tc_tiling_on_sc=False),
)
def _sc_gather(bt_hbm, fidx_hbm, out_hbm, idx_v, rows_v, sem):
    wid = lax.axis_index("s") * 2 + lax.axis_index("c")
    chunks_per_w = B // NW

    def body(c, carry):
        chunk = wid * chunks_per_w + c
        pltpu.sync_copy(fidx_hbm.at[pl.ds(chunk * IDX_ROWS, IDX_ROWS)], idx_v)
        handles = [
            pltpu.async_copy(bt_hbm.at[idx_v.at[j]], rows_v.at[j], sem)
            for j in range(IDX_ROWS)
        ]
        for h in handles:
            h.wait()
        pltpu.sync_copy(rows_v, out_hbm.at[chunk])
        return carry

    lax.fori_loop(0, chunks_per_w, body, 0)


BB = 16


def _epi_body(z_ref, o_ref):
    z = z_ref[...]
    lo = lax.bitcast_convert_type(z << 16, jnp.float32).astype(jnp.bfloat16)
    hi = lax.bitcast_convert_type(
        z & jnp.int32(-65536), jnp.float32
    ).astype(jnp.bfloat16)
    o_ref[...] = jnp.concatenate([lo, hi], axis=-1)


def _epilogue(z):
    return pl.pallas_call(
        _epi_body,
        grid=(B // BB,),
        in_specs=[pl.BlockSpec((BB, IDX_ROWS, 128, W32), lambda i: (i, 0, 0, 0))],
        out_specs=pl.BlockSpec((BB, IDX_ROWS, 128, EMB), lambda i: (i, 0, 0, 0)),
        out_shape=jax.ShapeDtypeStruct((B, IDX_ROWS, 128, EMB), jnp.bfloat16),
    )(z)


def kernel(inputs, table, W, b):
    btp, fidx = _prep(inputs, table, W, b.reshape(1, EMB))
    bt_i32 = btp.reshape(VOCAB * L, W32)
    fidx3 = fidx.reshape(ROWS // 128, 128)
    z = _sc_gather(bt_i32, fidx3)
    return _epilogue(z).reshape(B, L, EMB)
